# baseline (speedup 1.0000x reference)
import functools

import jax
import jax.numpy as jnp
from jax import lax
from jax.experimental import pallas as pl
from jax.experimental.pallas import tpu as pltpu
from jax.experimental.pallas import tpu_sc as plsc

_NUM_CORES = 2
_NUM_SUBCORES = 16
_NW = _NUM_CORES * _NUM_SUBCORES
_LANES = 16


def _sc_build_rel(tt, h, w):
    c = tt.shape[0]
    hw = h * w
    c_per_w = c // _NW
    mesh = plsc.VectorSubcoreMesh(core_axis_name="c", subcore_axis_name="s")

    @functools.partial(
        pl.kernel,
        out_type=jax.ShapeDtypeStruct((c, hw), jnp.float32),
        mesh=mesh,
        scratch_types=[
            pltpu.VMEM((c_per_w, 128), jnp.float32),
            pltpu.VMEM((c_per_w, hw), jnp.float32),
        ],
    )
    def rel_kernel(tt_hbm, rel_hbm, cols_v, rows_v):
        wid = lax.axis_index("s") * _NUM_CORES + lax.axis_index("c")
        base = wid * c_per_w
        pltpu.sync_copy(tt_hbm.at[pl.ds(base, c_per_w)], cols_v)
        for ci in range(c_per_w):

            def body(hh, carry):
                off = (w - 1) - hh
                for k in range(w // _LANES):
                    rows_v[ci, pl.ds(hh * w + k * _LANES, _LANES)] = (
                        cols_v[ci, pl.ds(off + k * _LANES, _LANES)]
                    )
                return carry

            lax.fori_loop(0, h, body, 0)
        pltpu.sync_copy(rows_v, rel_hbm.at[pl.ds(base, c_per_w)])

    return rel_kernel(tt)


def _tc_add_body(x_ref, rel_ref, o_ref):
    o_ref[...] = x_ref[...] + rel_ref[...]


def _tc_add(x3, rel):
    b, c, hw = x3.shape
    tile_c = 64
    grid = (c // tile_c, b)
    return pl.pallas_call(
        _tc_add_body,
        grid=grid,
        in_specs=[
            pl.BlockSpec((1, tile_c, hw), lambda ci, bi: (bi, ci, 0)),
            pl.BlockSpec((tile_c, hw), lambda ci, bi: (ci, 0)),
        ],
        out_specs=pl.BlockSpec((1, tile_c, hw), lambda ci, bi: (bi, ci, 0)),
        out_shape=jax.ShapeDtypeStruct((b, c, hw), jnp.float32),
    )(x3, rel)


def kernel(x, rel_emb_x, rel_emb_y):
    b, c, h, w = x.shape
    t = jnp.concatenate([rel_emb_x, rel_emb_y], axis=1)
    tt = jnp.pad(t.T, ((0, 0), (0, 128 - t.shape[0])))
    rel = _sc_build_rel(tt, h, w)
    out = _tc_add(x.reshape(b, c, h * w), rel)
    return out.reshape(b, c, h, w)


# speedup vs baseline: 2.4290x; 2.4290x over previous
import functools

import jax
import jax.numpy as jnp
from jax import lax
from jax.experimental import pallas as pl
from jax.experimental.pallas import tpu as pltpu
from jax.experimental.pallas import tpu_sc as plsc

_NUM_CORES = 2
_NUM_SUBCORES = 16
_NW = _NUM_CORES * _NUM_SUBCORES
_LANES = 16


def _sc_gather_rel(t_pad, h, w):
    c = t_pad.shape[1]
    hw = h * w
    rows_per_tile = hw // _NW
    mesh = plsc.VectorSubcoreMesh(core_axis_name="c", subcore_axis_name="s")

    @functools.partial(
        pl.kernel,
        out_type=jax.ShapeDtypeStruct((hw, c), jnp.float32),
        mesh=mesh,
        scratch_types=[
            pltpu.VMEM((rows_per_tile,), jnp.int32),
            pltpu.VMEM((rows_per_tile, c), jnp.float32),
            pltpu.SemaphoreType.DMA,
        ],
    )
    def rel_kernel(t_hbm, rel_hbm, idx_v, rows_v, sem):
        wid = lax.axis_index("s") * _NUM_CORES + lax.axis_index("c")
        base = wid * rows_per_tile
        lane = lax.iota(jnp.int32, _LANES)
        for k in range(rows_per_tile // _LANES):
            pos = base + k * _LANES + lane
            hh = jnp.right_shift(pos, w.bit_length() - 1)
            ww = jnp.bitwise_and(pos, w - 1)
            idx_v[pl.ds(k * _LANES, _LANES)] = ww - hh + (w - 1)
        pltpu.async_copy(t_hbm.at[idx_v], rows_v, sem).wait()
        pltpu.sync_copy(rows_v, rel_hbm.at[pl.ds(base, rows_per_tile)])

    return rel_kernel(t_pad)


def _tc_add_body(x_ref, rel_ref, o_ref):
    o_ref[...] = x_ref[...] + rel_ref[...]


def _tc_add(xt, rel):
    b, h, w, c = xt.shape
    tile_h = 16
    grid = (h // tile_h, b)
    return pl.pallas_call(
        _tc_add_body,
        grid=grid,
        in_specs=[
            pl.BlockSpec((1, tile_h, w, c), lambda hi, bi: (bi, hi, 0, 0)),
            pl.BlockSpec((tile_h, w, c), lambda hi, bi: (hi, 0, 0)),
        ],
        out_specs=pl.BlockSpec((1, tile_h, w, c), lambda hi, bi: (bi, hi, 0, 0)),
        out_shape=jax.ShapeDtypeStruct((b, h, w, c), jnp.float32),
    )(xt, rel)


def kernel(x, rel_emb_x, rel_emb_y):
    b, c, h, w = x.shape
    t = jnp.concatenate([rel_emb_x, rel_emb_y], axis=1)
    t_pad = jnp.pad(t, ((0, 1), (0, 0)))
    rel = _sc_gather_rel(t_pad, h, w).reshape(h, w, c)
    xt = jnp.transpose(x, (0, 2, 3, 1))
    out = _tc_add(xt, rel)
    return jnp.transpose(out, (0, 3, 1, 2))


# speedup vs baseline: 3.4153x; 1.4060x over previous
import functools

import jax
import jax.numpy as jnp
from jax import lax
from jax.experimental import pallas as pl
from jax.experimental.pallas import tpu as pltpu
from jax.experimental.pallas import tpu_sc as plsc

_NUM_CORES = 2
_NUM_SUBCORES = 16
_NW = _NUM_CORES * _NUM_SUBCORES
_LANES = 16


def _sc_gather_rel(t_pad, h, w):
    c = t_pad.shape[1]
    hw = h * w
    rows_per_tile = hw // _NW
    mesh = plsc.VectorSubcoreMesh(core_axis_name="c", subcore_axis_name="s")

    @functools.partial(
        pl.kernel,
        out_type=jax.ShapeDtypeStruct((hw, c), jnp.float32),
        mesh=mesh,
        scratch_types=[
            pltpu.VMEM((rows_per_tile,), jnp.int32),
            pltpu.VMEM((rows_per_tile, c), jnp.float32),
            pltpu.SemaphoreType.DMA,
        ],
    )
    def rel_kernel(t_hbm, rel_hbm, idx_v, rows_v, sem):
        wid = lax.axis_index("s") * _NUM_CORES + lax.axis_index("c")
        base = wid * rows_per_tile
        lane = lax.iota(jnp.int32, _LANES)
        for k in range(rows_per_tile // _LANES):
            pos = base + k * _LANES + lane
            hh = jnp.right_shift(pos, w.bit_length() - 1)
            ww = jnp.bitwise_and(pos, w - 1)
            idx_v[pl.ds(k * _LANES, _LANES)] = ww - hh + (w - 1)
        pltpu.async_copy(t_hbm.at[idx_v], rows_v, sem).wait()
        pltpu.sync_copy(rows_v, rel_hbm.at[pl.ds(base, rows_per_tile)])

    return rel_kernel(t_pad)


def _tc_add_body(x_ref, rel_ref, o_ref):
    o_ref[...] = x_ref[...] + rel_ref[...]


def _tc_add(xt, rel):
    b, h, w, c = xt.shape
    tile_h = 64
    grid = (h // tile_h, b)
    return pl.pallas_call(
        _tc_add_body,
        grid=grid,
        in_specs=[
            pl.BlockSpec((1, tile_h, w, c), lambda hi, bi: (bi, hi, 0, 0)),
            pl.BlockSpec((tile_h, w, c), lambda hi, bi: (hi, 0, 0)),
        ],
        out_specs=pl.BlockSpec((1, tile_h, w, c), lambda hi, bi: (bi, hi, 0, 0)),
        out_shape=jax.ShapeDtypeStruct((b, h, w, c), jnp.float32),
    )(xt, rel)


def kernel(x, rel_emb_x, rel_emb_y):
    b, c, h, w = x.shape
    t = jnp.concatenate([rel_emb_x, rel_emb_y], axis=1)
    t_pad = jnp.pad(t, ((0, 1), (0, 0)))
    rel = _sc_gather_rel(t_pad, h, w).reshape(h, w, c)
    xt = jnp.transpose(x, (0, 2, 3, 1))
    out = _tc_add(xt, rel)
    return jnp.transpose(out, (0, 3, 1, 2))


# speedup vs baseline: 3.4999x; 1.0248x over previous
import functools

import jax
import jax.numpy as jnp
from jax import lax
from jax.experimental import pallas as pl
from jax.experimental.pallas import tpu as pltpu
from jax.experimental.pallas import tpu_sc as plsc

_NUM_CORES = 2
_NUM_SUBCORES = 16
_NW = _NUM_CORES * _NUM_SUBCORES
_LANES = 16


def _sc_gather_rel(t_pad, h, w):
    c = t_pad.shape[1]
    hw = h * w
    rows_per_tile = hw // _NW
    mesh = plsc.VectorSubcoreMesh(core_axis_name="c", subcore_axis_name="s")

    @functools.partial(
        pl.kernel,
        out_type=jax.ShapeDtypeStruct((hw, c), jnp.float32),
        mesh=mesh,
        scratch_types=[
            pltpu.VMEM((rows_per_tile,), jnp.int32),
            pltpu.VMEM((rows_per_tile, c), jnp.float32),
            pltpu.SemaphoreType.DMA,
        ],
    )
    def rel_kernel(t_hbm, rel_hbm, idx_v, rows_v, sem):
        wid = lax.axis_index("s") * _NUM_CORES + lax.axis_index("c")
        base = wid * rows_per_tile
        lane = lax.iota(jnp.int32, _LANES)
        for k in range(rows_per_tile // _LANES):
            pos = base + k * _LANES + lane
            hh = jnp.right_shift(pos, w.bit_length() - 1)
            ww = jnp.bitwise_and(pos, w - 1)
            idx_v[pl.ds(k * _LANES, _LANES)] = ww - hh + (w - 1)
        pltpu.async_copy(t_hbm.at[idx_v], rows_v, sem).wait()
        pltpu.sync_copy(rows_v, rel_hbm.at[pl.ds(base, rows_per_tile)])

    return rel_kernel(t_pad)


def _tc_add_body(x_ref, rel_ref, o_ref):
    o_ref[...] = x_ref[...] + rel_ref[...]


def _tc_add(xt, rel):
    b, h, w, c = xt.shape
    tile_b = 2
    grid = (b // tile_b,)
    return pl.pallas_call(
        _tc_add_body,
        grid=grid,
        in_specs=[
            pl.BlockSpec((tile_b, h, w, c), lambda bi: (bi, 0, 0, 0)),
            pl.BlockSpec((h, w, c), lambda bi: (0, 0, 0)),
        ],
        out_specs=pl.BlockSpec((tile_b, h, w, c), lambda bi: (bi, 0, 0, 0)),
        out_shape=jax.ShapeDtypeStruct((b, h, w, c), jnp.float32),
    )(xt, rel)


def kernel(x, rel_emb_x, rel_emb_y):
    b, c, h, w = x.shape
    t = jnp.concatenate([rel_emb_x, rel_emb_y], axis=1)
    t_pad = jnp.pad(t, ((0, 1), (0, 0)))
    rel = _sc_gather_rel(t_pad, h, w).reshape(h, w, c)
    xt = jnp.transpose(x, (0, 2, 3, 1))
    out = _tc_add(xt, rel)
    return jnp.transpose(out, (0, 3, 1, 2))
